# Initial kernel scaffold; baseline (speedup 1.0000x reference)
import jax
import jax.numpy as jnp
from jax.experimental import pallas as pl


def kernel(x, row_embed, col_embed):
    raise NotImplementedError("write your pallas kernel here")


# speedup vs baseline: 1.0036x; 1.0036x over previous
import jax
import jax.numpy as jnp
from jax.experimental import pallas as pl


def _pos_kernel(row_ref, col_ref, out_ref):
    col = col_ref[...]
    row = row_ref[...]
    h, d = row.shape
    w = col.shape[0]
    out_ref[0, :, :, 0:d] = jnp.broadcast_to(col[None, :, :], (h, w, d))
    out_ref[0, :, :, d : 2 * d] = jnp.broadcast_to(row[:, None, :], (h, w, d))


def kernel(x, row_embed, col_embed):
    b = x.shape[0]
    h, w = x.shape[-3], x.shape[-2]
    d = row_embed.shape[-1]
    out = pl.pallas_call(
        _pos_kernel,
        grid=(b,),
        in_specs=[
            pl.BlockSpec((h, d), lambda i: (0, 0)),
            pl.BlockSpec((w, d), lambda i: (0, 0)),
        ],
        out_specs=pl.BlockSpec((1, h, w, 2 * d), lambda i: (i, 0, 0, 0)),
        out_shape=jax.ShapeDtypeStruct((b, h, w, 2 * d), row_embed.dtype),
    )(row_embed, col_embed)
    return out.reshape(b, h * w, 2 * d)


# speedup vs baseline: 1.1267x; 1.1226x over previous
import jax
import jax.numpy as jnp
from jax.experimental import pallas as pl
from jax.experimental.pallas import tpu as pltpu


def _pos_kernel(row_ref, col_ref, out_hbm, tile_ref, sem):
    h, d = row_ref.shape
    w = col_ref.shape[0]
    b = out_hbm.shape[0]
    tile_ref[:, :, 0:d] = jnp.broadcast_to(col_ref[...][None, :, :], (h, w, d))
    tile_ref[:, :, d : 2 * d] = jnp.broadcast_to(row_ref[...][:, None, :], (h, w, d))
    copies = [
        pltpu.make_async_copy(tile_ref, out_hbm.at[i], sem.at[i]) for i in range(b)
    ]
    for c in copies:
        c.start()
    for c in copies:
        c.wait()


def kernel(x, row_embed, col_embed):
    b = x.shape[0]
    h, w = x.shape[-3], x.shape[-2]
    d = row_embed.shape[-1]
    out = pl.pallas_call(
        _pos_kernel,
        in_specs=[
            pl.BlockSpec(memory_space=pltpu.MemorySpace.VMEM),
            pl.BlockSpec(memory_space=pltpu.MemorySpace.VMEM),
        ],
        out_specs=pl.BlockSpec(memory_space=pltpu.MemorySpace.HBM),
        out_shape=jax.ShapeDtypeStruct((b, h, w, 2 * d), row_embed.dtype),
        scratch_shapes=[
            pltpu.VMEM((h, w, 2 * d), row_embed.dtype),
            pltpu.SemaphoreType.DMA((b,)),
        ],
    )(row_embed, col_embed)
    return out.reshape(b, h * w, 2 * d)
